# Initial kernel scaffold; baseline (speedup 1.0000x reference)
import jax
import jax.numpy as jnp
from jax.experimental import pallas as pl


def kernel(x, edge_index, W1, b1, W2, b2):
    raise NotImplementedError("write your pallas kernel here")


# speedup vs baseline: 45.0506x; 45.0506x over previous
import functools

import jax
import jax.numpy as jnp
from jax import lax
from jax.experimental import pallas as pl
from jax.experimental.pallas import tpu as pltpu
from jax.experimental.pallas import tpu_sc as plsc

N = 10000
E = 320000
D_IN = 128
D_HID = 64
D_OUT = 10
D_OUT_PAD = 16

NC = 2
NS = 16
NW = NC * NS
CH = 128
NCH = 80
E_PAD = NW * NCH * CH

RPT = 632
N_ACC = RPT * NS
DPT = 640
N_DEG = DPT * NS


def _mesh():
    return plsc.VectorSubcoreMesh(core_axis_name="c", subcore_axis_name="s")


@functools.partial(
    pl.kernel,
    mesh=_mesh(),
    out_type=jax.ShapeDtypeStruct((NC, N_DEG), jnp.float32),
    scratch_types=[
        pltpu.VMEM((NCH, CH), jnp.int32),
        pltpu.VMEM((CH,), jnp.float32),
        pltpu.VMEM((DPT,), jnp.float32),
        pltpu.VMEM_SHARED((N_DEG,), jnp.float32),
    ],
)
def _deg_kernel(dsts_hbm, out_hbm, idx_v, ones_v, zb_v, acc_sh):
    cid = lax.axis_index("c")
    sid = lax.axis_index("s")
    w = sid * NC + cid
    pltpu.sync_copy(dsts_hbm.at[w], idx_v)
    for i in range(CH // 16):
        ones_v[pl.ds(i * 16, 16)] = jnp.ones((16,), jnp.float32)
    for i in range(DPT // 16):
        zb_v[pl.ds(i * 16, 16)] = jnp.zeros((16,), jnp.float32)
    pltpu.sync_copy(zb_v, acc_sh.at[pl.ds(sid * DPT, DPT)])
    plsc.subcore_barrier()

    def body(j, carry):
        pltpu.sync_copy(ones_v, acc_sh.at[idx_v.at[j]], add=True)
        return carry

    lax.fori_loop(0, NCH, body, 0)
    plsc.subcore_barrier()
    pltpu.sync_copy(acc_sh.at[pl.ds(sid * DPT, DPT)], zb_v)
    pltpu.sync_copy(zb_v, out_hbm.at[cid, pl.ds(sid * DPT, DPT)])


def _make_agg(d):
    @functools.partial(
        pl.kernel,
        mesh=_mesh(),
        compiler_params=pltpu.CompilerParams(use_tc_tiling_on_sc=False),
        out_type=jax.ShapeDtypeStruct((NC, N_ACC, d), jnp.float32),
        scratch_types=[
            pltpu.VMEM((NCH, CH), jnp.int32),
            pltpu.VMEM((NCH, CH), jnp.int32),
            pltpu.VMEM((CH, d), jnp.float32),
            pltpu.VMEM((CH, d), jnp.float32),
            pltpu.VMEM((RPT, d), jnp.float32),
            pltpu.VMEM_SHARED((N_ACC, d), jnp.float32),
            pltpu.SemaphoreType.DMA,
            pltpu.SemaphoreType.DMA,
        ],
    )
    def agg(h_hbm, srcs_hbm, dsts_hbm, out_hbm,
            src_v, dst_v, rows0, rows1, zb_v, acc_sh, sem0, sem1):
        cid = lax.axis_index("c")
        sid = lax.axis_index("s")
        w = sid * NC + cid
        pltpu.sync_copy(srcs_hbm.at[w], src_v)
        pltpu.sync_copy(dsts_hbm.at[w], dst_v)

        def zbody(i, carry):
            for k in range(d // 16):
                zb_v[i, pl.ds(k * 16, 16)] = jnp.zeros((16,), jnp.float32)
            return carry

        lax.fori_loop(0, RPT, zbody, 0)
        pltpu.sync_copy(zb_v, acc_sh.at[pl.ds(sid * RPT, RPT)])
        plsc.subcore_barrier()

        rows = (rows0, rows1)
        sems = (sem0, sem1)
        for b in range(2):
            pltpu.async_copy(h_hbm.at[src_v.at[b]], rows[b], sems[b])

        def body(g, carry):
            for b in range(2):
                c = g * 2 + b
                pltpu.make_async_copy(
                    h_hbm.at[src_v.at[c]], rows[b], sems[b]).wait()
                pltpu.sync_copy(rows[b], acc_sh.at[dst_v.at[c]], add=True)
                nc_ = c + 2

                @pl.when(nc_ < NCH)
                def _():
                    pltpu.async_copy(
                        h_hbm.at[src_v.at[nc_]], rows[b], sems[b])
            return carry

        lax.fori_loop(0, NCH // 2, body, 0)
        plsc.subcore_barrier()
        pltpu.sync_copy(acc_sh.at[pl.ds(sid * RPT, RPT)], zb_v)
        pltpu.sync_copy(zb_v, out_hbm.at[cid, pl.ds(sid * RPT, RPT)])

    return agg


_agg64 = _make_agg(D_HID)
_agg16 = _make_agg(D_OUT_PAD)


def _k2_body(deg_ref, x_ref, w1_ref, dinv_ref, h1_ref):
    deg = deg_ref[0, :N] + deg_ref[1, :N] + 1.0
    dinv = lax.rsqrt(deg)
    h = jnp.dot(x_ref[...], w1_ref[...],
                preferred_element_type=jnp.float32,
                precision=lax.Precision.HIGHEST)
    dinv_ref[...] = dinv
    h1_ref[...] = h * dinv[:, None]


def _k4_body(s1_ref, h1_ref, dinv_ref, b1_ref, w2_ref, h2_ref):
    s = s1_ref[0, :N, :] + s1_ref[1, :N, :] + h1_ref[...]
    dinv = dinv_ref[...]
    o1 = jnp.maximum(s * dinv[:, None] + b1_ref[...][None, :], 0.0)
    h2_ref[...] = jnp.dot(o1 * dinv[:, None], w2_ref[...],
                          preferred_element_type=jnp.float32,
                          precision=lax.Precision.HIGHEST)


def _k6_body(s2_ref, h2_ref, dinv_ref, b2_ref, out_ref):
    s = s2_ref[0, :N, :] + s2_ref[1, :N, :] + h2_ref[...]
    out_ref[...] = s * dinv_ref[...][:, None] + b2_ref[...][None, :]


def kernel(x, edge_index, W1, b1, W2, b2):
    src = edge_index[0]
    dst = edge_index[1]
    pad = E_PAD - E
    ar = jnp.arange(pad, dtype=jnp.int32)
    pad_src = ar % N
    pad_dst = N + (ar % 16)
    srcs = jnp.concatenate([src, pad_src]).reshape(NW, NCH, CH)
    dsts = jnp.concatenate([dst, pad_dst]).reshape(NW, NCH, CH)

    deg2 = _deg_kernel(dsts)

    dinv, h1 = pl.pallas_call(
        _k2_body,
        out_shape=[
            jax.ShapeDtypeStruct((N,), jnp.float32),
            jax.ShapeDtypeStruct((N, D_HID), jnp.float32),
        ],
    )(deg2, x, W1)

    s1 = _agg64(h1, srcs, dsts)

    w2p = jnp.pad(W2, ((0, 0), (0, D_OUT_PAD - D_OUT)))
    b2p = jnp.pad(b2, (0, D_OUT_PAD - D_OUT))
    h2 = pl.pallas_call(
        _k4_body,
        out_shape=jax.ShapeDtypeStruct((N, D_OUT_PAD), jnp.float32),
    )(s1, h1, dinv, b1, w2p)

    s2 = _agg16(h2, srcs, dsts)

    out = pl.pallas_call(
        _k6_body,
        out_shape=jax.ShapeDtypeStruct((N, D_OUT_PAD), jnp.float32),
    )(s2, h2, dinv, b2p)

    return out[:, :D_OUT]


# speedup vs baseline: 46.3953x; 1.0298x over previous
import functools

import jax
import jax.numpy as jnp
from jax import lax
from jax.experimental import pallas as pl
from jax.experimental.pallas import tpu as pltpu
from jax.experimental.pallas import tpu_sc as plsc

N = 10000
E = 320000
D_IN = 128
D_HID = 64
D_OUT = 10
D_OUT_PAD = 16

NC = 2
NS = 16
NW = NC * NS
CH = 128
NCH = 80
E_PAD = NW * NCH * CH

RPT = 632
N_ACC = RPT * NS
DPT = 640
N_DEG = DPT * NS


def _mesh():
    return plsc.VectorSubcoreMesh(core_axis_name="c", subcore_axis_name="s")


@functools.partial(
    pl.kernel,
    mesh=_mesh(),
    out_type=jax.ShapeDtypeStruct((NC, N_DEG), jnp.float32),
    scratch_types=[
        pltpu.VMEM((NCH, CH), jnp.int32),
        pltpu.VMEM((CH,), jnp.float32),
        pltpu.VMEM((DPT,), jnp.float32),
        pltpu.VMEM_SHARED((N_DEG,), jnp.float32),
    ],
)
def _deg_kernel(dsts_hbm, out_hbm, idx_v, ones_v, zb_v, acc_sh):
    cid = lax.axis_index("c")
    sid = lax.axis_index("s")
    w = sid * NC + cid
    pltpu.sync_copy(dsts_hbm.at[w], idx_v)
    for i in range(CH // 16):
        ones_v[pl.ds(i * 16, 16)] = jnp.ones((16,), jnp.float32)
    for i in range(DPT // 16):
        zb_v[pl.ds(i * 16, 16)] = jnp.zeros((16,), jnp.float32)
    pltpu.sync_copy(zb_v, acc_sh.at[pl.ds(sid * DPT, DPT)])
    plsc.subcore_barrier()

    def body(j, carry):
        pltpu.sync_copy(ones_v, acc_sh.at[idx_v.at[j]], add=True)
        return carry

    lax.fori_loop(0, NCH, body, 0)
    plsc.subcore_barrier()
    pltpu.sync_copy(acc_sh.at[pl.ds(sid * DPT, DPT)], zb_v)
    pltpu.sync_copy(zb_v, out_hbm.at[cid, pl.ds(sid * DPT, DPT)])


def _make_agg(d):
    @functools.partial(
        pl.kernel,
        mesh=_mesh(),
        compiler_params=pltpu.CompilerParams(use_tc_tiling_on_sc=False),
        out_type=jax.ShapeDtypeStruct((NC, N_ACC, d), jnp.float32),
        scratch_types=[
            pltpu.VMEM((NCH, CH), jnp.int32),
            pltpu.VMEM((NCH, CH), jnp.int32),
            pltpu.VMEM((CH, d), jnp.float32),
            pltpu.VMEM((CH, d), jnp.float32),
            pltpu.VMEM((RPT, d), jnp.float32),
            pltpu.VMEM_SHARED((N_ACC, d), jnp.float32),
            pltpu.SemaphoreType.DMA,
            pltpu.SemaphoreType.DMA,
        ],
    )
    def agg(h_hbm, srcs_hbm, dsts_hbm, out_hbm,
            src_v, dst_v, rows0, rows1, zb_v, acc_sh, sem0, sem1):
        cid = lax.axis_index("c")
        sid = lax.axis_index("s")
        w = sid * NC + cid
        pltpu.sync_copy(srcs_hbm.at[w], src_v)
        pltpu.sync_copy(dsts_hbm.at[w], dst_v)

        def zbody(i, carry):
            for k in range(d // 16):
                zb_v[i, pl.ds(k * 16, 16)] = jnp.zeros((16,), jnp.float32)
            return carry

        lax.fori_loop(0, RPT, zbody, 0)
        pltpu.sync_copy(zb_v, acc_sh.at[pl.ds(sid * RPT, RPT)])
        plsc.subcore_barrier()

        rows = (rows0, rows1)
        sems = (sem0, sem1)
        for b in range(2):
            pltpu.async_copy(h_hbm.at[src_v.at[b]], rows[b], sems[b])

        def body(g, carry):
            for b in range(2):
                c = g * 2 + b
                pltpu.make_async_copy(
                    h_hbm.at[src_v.at[c]], rows[b], sems[b]).wait()
                pltpu.sync_copy(rows[b], acc_sh.at[dst_v.at[c]], add=True)
                nc_ = c + 2

                @pl.when(nc_ < NCH)
                def _():
                    pltpu.async_copy(
                        h_hbm.at[src_v.at[nc_]], rows[b], sems[b])
            return carry

        lax.fori_loop(0, NCH // 2, body, 0)
        plsc.subcore_barrier()
        pltpu.sync_copy(acc_sh.at[pl.ds(sid * RPT, RPT)], zb_v)
        pltpu.sync_copy(zb_v, out_hbm.at[cid, pl.ds(sid * RPT, RPT)])

    return agg


_agg64 = _make_agg(D_HID)
_agg16 = _make_agg(D_OUT_PAD)


def _k2_body(deg_ref, x_ref, w1_ref, dinv_ref, h1_ref):
    deg = deg_ref[0, :N] + deg_ref[1, :N] + 1.0
    dinv = lax.rsqrt(deg)
    h = jnp.dot(x_ref[...], w1_ref[...],
                preferred_element_type=jnp.float32)
    dinv_ref[...] = dinv
    h1_ref[...] = h * dinv[:, None]


def _k4_body(s1_ref, h1_ref, dinv_ref, b1_ref, w2_ref, h2_ref):
    s = s1_ref[0, :N, :] + s1_ref[1, :N, :] + h1_ref[...]
    dinv = dinv_ref[...]
    o1 = jnp.maximum(s * dinv[:, None] + b1_ref[...][None, :], 0.0)
    h2_ref[...] = jnp.dot(o1 * dinv[:, None], w2_ref[...],
                          preferred_element_type=jnp.float32)


def _k6_body(s2_ref, h2_ref, dinv_ref, b2_ref, out_ref):
    s = s2_ref[0, :N, :D_OUT] + s2_ref[1, :N, :D_OUT] + h2_ref[:, :D_OUT]
    out_ref[...] = s * dinv_ref[...][:, None] + b2_ref[...][None, :]


def kernel(x, edge_index, W1, b1, W2, b2):
    src = edge_index[0]
    dst = edge_index[1]
    pad = E_PAD - E
    ar = jnp.arange(pad, dtype=jnp.int32)
    pad_src = ar & 8191
    pad_dst = N + (ar & 15)
    srcs = jnp.concatenate([src, pad_src]).reshape(NW, NCH, CH)
    dsts = jnp.concatenate([dst, pad_dst]).reshape(NW, NCH, CH)

    deg2 = _deg_kernel(dsts)

    dinv, h1 = pl.pallas_call(
        _k2_body,
        out_shape=[
            jax.ShapeDtypeStruct((N,), jnp.float32),
            jax.ShapeDtypeStruct((N, D_HID), jnp.float32),
        ],
    )(deg2, x, W1)

    s1 = _agg64(h1, srcs, dsts)

    w2p = jnp.pad(W2, ((0, 0), (0, D_OUT_PAD - D_OUT)))
    h2 = pl.pallas_call(
        _k4_body,
        out_shape=jax.ShapeDtypeStruct((N, D_OUT_PAD), jnp.float32),
    )(s1, h1, dinv, b1, w2p)

    s2 = _agg16(h2, srcs, dsts)

    out = pl.pallas_call(
        _k6_body,
        out_shape=jax.ShapeDtypeStruct((N, D_OUT), jnp.float32),
    )(s2, h2, dinv, b2)

    return out


# speedup vs baseline: 47.6814x; 1.0277x over previous
import functools

import jax
import jax.numpy as jnp
from jax import lax
from jax.experimental import pallas as pl
from jax.experimental.pallas import tpu as pltpu
from jax.experimental.pallas import tpu_sc as plsc

N = 10000
E = 320000
D_IN = 128
D_HID = 64
D_OUT = 10
D_OUT_PAD = 16

NC = 2
NS = 16
NW = NC * NS
CH = 128
NCHUNKS = E // CH
CPT = NCHUNKS // NW
REM = NCHUNKS - CPT * NW
NCH = CPT + 1

RPT = 632
N_ACC = RPT * NS
DPT = 640
N_DEG = DPT * NS


def _mesh():
    return plsc.VectorSubcoreMesh(core_axis_name="c", subcore_axis_name="s")


@functools.partial(
    pl.kernel,
    mesh=_mesh(),
    compiler_params=pltpu.CompilerParams(use_tc_tiling_on_sc=False),
    out_type=jax.ShapeDtypeStruct((NC, N_DEG), jnp.float32),
    scratch_types=[
        pltpu.VMEM((NCH, CH), jnp.int32),
        pltpu.VMEM((CH,), jnp.float32),
        pltpu.VMEM((DPT,), jnp.float32),
        pltpu.VMEM_SHARED((N_DEG,), jnp.float32),
    ],
)
def _deg_kernel(ei3_hbm, out_hbm, idx_v, ones_v, zb_v, acc_sh):
    cid = lax.axis_index("c")
    sid = lax.axis_index("s")
    w = sid * NC + cid
    pltpu.sync_copy(ei3_hbm.at[1, pl.ds(w * CPT, CPT)],
                    idx_v.at[pl.ds(0, CPT)])

    @pl.when(w < REM)
    def _():
        pltpu.sync_copy(ei3_hbm.at[1, pl.ds(NW * CPT + w, 1)],
                        idx_v.at[pl.ds(CPT, 1)])

    nch = CPT + jnp.where(w < REM, 1, 0)
    for i in range(CH // 16):
        ones_v[pl.ds(i * 16, 16)] = jnp.ones((16,), jnp.float32)
    for i in range(DPT // 16):
        zb_v[pl.ds(i * 16, 16)] = jnp.zeros((16,), jnp.float32)
    pltpu.sync_copy(zb_v, acc_sh.at[pl.ds(sid * DPT, DPT)])
    plsc.subcore_barrier()

    def body(j, carry):
        pltpu.sync_copy(ones_v, acc_sh.at[idx_v.at[j]], add=True)
        return carry

    lax.fori_loop(0, nch, body, 0)
    plsc.subcore_barrier()
    pltpu.sync_copy(acc_sh.at[pl.ds(sid * DPT, DPT)], zb_v)
    pltpu.sync_copy(zb_v, out_hbm.at[cid, pl.ds(sid * DPT, DPT)])


def _make_agg(d):
    @functools.partial(
        pl.kernel,
        mesh=_mesh(),
        compiler_params=pltpu.CompilerParams(use_tc_tiling_on_sc=False),
        out_type=jax.ShapeDtypeStruct((NC, N_ACC, d), jnp.float32),
        scratch_types=[
            pltpu.VMEM((NCH, CH), jnp.int32),
            pltpu.VMEM((NCH, CH), jnp.int32),
            pltpu.VMEM((CH, d), jnp.float32),
            pltpu.VMEM((CH, d), jnp.float32),
            pltpu.VMEM((RPT, d), jnp.float32),
            pltpu.VMEM_SHARED((N_ACC, d), jnp.float32),
            pltpu.SemaphoreType.DMA,
            pltpu.SemaphoreType.DMA,
        ],
    )
    def agg(h_hbm, ei3_hbm, zeros_hbm, out_hbm,
            src_v, dst_v, rows0, rows1, zb_v, acc_sh, sem0, sem1):
        cid = lax.axis_index("c")
        sid = lax.axis_index("s")
        w = sid * NC + cid
        pltpu.sync_copy(ei3_hbm.at[0, pl.ds(w * CPT, CPT)],
                        src_v.at[pl.ds(0, CPT)])
        pltpu.sync_copy(ei3_hbm.at[1, pl.ds(w * CPT, CPT)],
                        dst_v.at[pl.ds(0, CPT)])

        @pl.when(w < REM)
        def _():
            pltpu.sync_copy(ei3_hbm.at[0, pl.ds(NW * CPT + w, 1)],
                            src_v.at[pl.ds(CPT, 1)])
            pltpu.sync_copy(ei3_hbm.at[1, pl.ds(NW * CPT + w, 1)],
                            dst_v.at[pl.ds(CPT, 1)])

        nch = CPT + jnp.where(w < REM, 1, 0)
        pltpu.sync_copy(zeros_hbm, zb_v)
        pltpu.sync_copy(zb_v, acc_sh.at[pl.ds(sid * RPT, RPT)])
        plsc.subcore_barrier()

        rows = (rows0, rows1)
        sems = (sem0, sem1)
        for b in range(2):
            pltpu.async_copy(h_hbm.at[src_v.at[b]], rows[b], sems[b])

        def body(g, carry):
            for b in range(2):
                c = g * 2 + b
                pltpu.make_async_copy(
                    h_hbm.at[src_v.at[c]], rows[b], sems[b]).wait()
                pltpu.sync_copy(rows[b], acc_sh.at[dst_v.at[c]], add=True)
                nc_ = c + 2

                @pl.when(nc_ < nch)
                def _():
                    pltpu.async_copy(
                        h_hbm.at[src_v.at[nc_]], rows[b], sems[b])
            return carry

        lax.fori_loop(0, CPT // 2, body, 0)

        @pl.when(w < REM)
        def _():
            pltpu.make_async_copy(
                h_hbm.at[src_v.at[CPT]], rows0, sem0).wait()
            pltpu.sync_copy(rows0, acc_sh.at[dst_v.at[CPT]], add=True)

        plsc.subcore_barrier()
        pltpu.sync_copy(acc_sh.at[pl.ds(sid * RPT, RPT)], zb_v)
        pltpu.sync_copy(zb_v, out_hbm.at[cid, pl.ds(sid * RPT, RPT)])

    return agg


_agg64 = _make_agg(D_HID)
_agg16 = _make_agg(D_OUT_PAD)


def _k2_body(deg_ref, x_ref, w1_ref, dinv_ref, h1_ref):
    deg = deg_ref[0, :N] + deg_ref[1, :N] + 1.0
    dinv = lax.rsqrt(deg)
    h = jnp.dot(x_ref[...], w1_ref[...],
                preferred_element_type=jnp.float32)
    dinv_ref[...] = dinv
    h1_ref[...] = h * dinv[:, None]


def _k4_body(s1_ref, h1_ref, dinv_ref, b1_ref, w2_ref, h2_ref):
    s = s1_ref[0, :N, :] + s1_ref[1, :N, :] + h1_ref[...]
    dinv = dinv_ref[...]
    o1 = jnp.maximum(s * dinv[:, None] + b1_ref[...][None, :], 0.0)
    h2_ref[...] = jnp.dot(o1 * dinv[:, None], w2_ref[...],
                          preferred_element_type=jnp.float32)


def _k6_body(s2_ref, h2_ref, dinv_ref, b2_ref, out_ref):
    s = s2_ref[0, :N, :D_OUT] + s2_ref[1, :N, :D_OUT] + h2_ref[:, :D_OUT]
    out_ref[...] = s * dinv_ref[...][:, None] + b2_ref[...][None, :]


def kernel(x, edge_index, W1, b1, W2, b2):
    ei3 = jnp.reshape(edge_index, (2, NCHUNKS, CH))
    z64 = jnp.zeros((RPT, D_HID), jnp.float32)
    z16 = jnp.zeros((RPT, D_OUT_PAD), jnp.float32)

    deg2 = _deg_kernel(ei3)

    dinv, h1 = pl.pallas_call(
        _k2_body,
        out_shape=[
            jax.ShapeDtypeStruct((N,), jnp.float32),
            jax.ShapeDtypeStruct((N, D_HID), jnp.float32),
        ],
    )(deg2, x, W1)

    s1 = _agg64(h1, ei3, z64)

    w2p = jnp.pad(W2, ((0, 0), (0, D_OUT_PAD - D_OUT)))
    h2 = pl.pallas_call(
        _k4_body,
        out_shape=jax.ShapeDtypeStruct((N, D_OUT_PAD), jnp.float32),
    )(s1, h1, dinv, b1, w2p)

    s2 = _agg16(h2, ei3, z16)

    out = pl.pallas_call(
        _k6_body,
        out_shape=jax.ShapeDtypeStruct((N, D_OUT), jnp.float32),
    )(s2, h2, dinv, b2)

    return out


# speedup vs baseline: 48.3167x; 1.0133x over previous
import functools

import jax
import jax.numpy as jnp
from jax import lax
from jax.experimental import pallas as pl
from jax.experimental.pallas import tpu as pltpu
from jax.experimental.pallas import tpu_sc as plsc

N = 10000
E = 320000
D_IN = 128
D_HID = 64
D_OUT = 10
D_OUT_PAD = 16

NC = 2
NS = 16
NW = NC * NS
CH = 128
NCHUNKS = E // CH
CPT = NCHUNKS // NW
REM = NCHUNKS - CPT * NW
NCH = CPT + 1

RPT = 632
N_ACC = RPT * NS
DPT = 640
N_DEG = DPT * NS


def _mesh():
    return plsc.VectorSubcoreMesh(core_axis_name="c", subcore_axis_name="s")


@functools.partial(
    pl.kernel,
    mesh=_mesh(),
    compiler_params=pltpu.CompilerParams(use_tc_tiling_on_sc=False),
    out_type=jax.ShapeDtypeStruct((NC, N_DEG), jnp.float32),
    scratch_types=[
        pltpu.VMEM((NCH, CH), jnp.int32),
        pltpu.VMEM((CH,), jnp.float32),
        pltpu.VMEM((DPT,), jnp.float32),
        pltpu.VMEM_SHARED((N_DEG,), jnp.float32),
    ],
)
def _deg_kernel(ei3_hbm, out_hbm, idx_v, ones_v, zb_v, acc_sh):
    cid = lax.axis_index("c")
    sid = lax.axis_index("s")
    w = sid * NC + cid
    pltpu.sync_copy(ei3_hbm.at[1, pl.ds(w * CPT, CPT)],
                    idx_v.at[pl.ds(0, CPT)])

    @pl.when(w < REM)
    def _():
        pltpu.sync_copy(ei3_hbm.at[1, pl.ds(NW * CPT + w, 1)],
                        idx_v.at[pl.ds(CPT, 1)])

    nch = CPT + jnp.where(w < REM, 1, 0)
    for i in range(CH // 16):
        ones_v[pl.ds(i * 16, 16)] = jnp.ones((16,), jnp.float32)
    for i in range(DPT // 16):
        zb_v[pl.ds(i * 16, 16)] = jnp.zeros((16,), jnp.float32)
    pltpu.sync_copy(zb_v, acc_sh.at[pl.ds(sid * DPT, DPT)])
    plsc.subcore_barrier()

    def body(j, carry):
        pltpu.sync_copy(ones_v, acc_sh.at[idx_v.at[j]], add=True)
        return carry

    lax.fori_loop(0, nch, body, 0)
    plsc.subcore_barrier()
    pltpu.sync_copy(acc_sh.at[pl.ds(sid * DPT, DPT)], zb_v)
    pltpu.sync_copy(zb_v, out_hbm.at[cid, pl.ds(sid * DPT, DPT)])


def _make_agg(d):
    @functools.partial(
        pl.kernel,
        mesh=_mesh(),
        compiler_params=pltpu.CompilerParams(use_tc_tiling_on_sc=False),
        out_type=jax.ShapeDtypeStruct((NC, N_ACC, d), jnp.float32),
        scratch_types=[
            pltpu.VMEM((NCH, CH), jnp.int32),
            pltpu.VMEM((NCH, CH), jnp.int32),
            pltpu.VMEM((CH, d), jnp.float32),
            pltpu.VMEM((CH, d), jnp.float32),
            pltpu.VMEM((CH, d), jnp.float32),
            pltpu.VMEM((CH, d), jnp.float32),
            pltpu.VMEM_SHARED((N_ACC, d), jnp.float32),
            pltpu.SemaphoreType.DMA,
            pltpu.SemaphoreType.DMA,
            pltpu.SemaphoreType.DMA,
            pltpu.SemaphoreType.DMA,
            pltpu.SemaphoreType.DMA,
            pltpu.SemaphoreType.DMA,
            pltpu.SemaphoreType.DMA,
            pltpu.SemaphoreType.DMA,
        ],
    )
    def agg(h_hbm, ei3_hbm, zeros_hbm, out_hbm,
            src_v, dst_v, rows0, rows1, rows2, rows3, acc_sh,
            g0, g1, g2, g3, s0, s1, s2, s3):
        cid = lax.axis_index("c")
        sid = lax.axis_index("s")
        w = sid * NC + cid
        pltpu.sync_copy(ei3_hbm.at[0, pl.ds(w * CPT, CPT)],
                        src_v.at[pl.ds(0, CPT)])
        pltpu.sync_copy(ei3_hbm.at[1, pl.ds(w * CPT, CPT)],
                        dst_v.at[pl.ds(0, CPT)])

        @pl.when(w < REM)
        def _():
            pltpu.sync_copy(ei3_hbm.at[0, pl.ds(NW * CPT + w, 1)],
                            src_v.at[pl.ds(CPT, 1)])
            pltpu.sync_copy(ei3_hbm.at[1, pl.ds(NW * CPT + w, 1)],
                            dst_v.at[pl.ds(CPT, 1)])

        nch = CPT + jnp.where(w < REM, 1, 0)
        pltpu.sync_copy(zeros_hbm, acc_sh.at[pl.ds(sid * RPT, RPT)])
        plsc.subcore_barrier()

        rows = (rows0, rows1, rows2, rows3)
        gs = (g0, g1, g2, g3)
        ss = (s0, s1, s2, s3)
        for b in range(2):
            pltpu.async_copy(h_hbm.at[src_v.at[b]], rows[b], gs[b])

        def body(g, carry):
            for b in range(4):
                c = g * 4 + b

                @pl.when(c < nch)
                def _():
                    bn = (b + 2) % 4
                    pltpu.make_async_copy(
                        h_hbm.at[src_v.at[c]], rows[b], gs[b]).wait()
                    pltpu.sync_copy(rows[b], acc_sh.at[dst_v.at[c]], add=True)

                    @pl.when(c + 2 < nch)
                    def _():
                        pltpu.async_copy(
                            h_hbm.at[src_v.at[c + 2]], rows[bn], gs[bn])
            return carry

        lax.fori_loop(0, (NCH + 3) // 4, body, 0)

        plsc.subcore_barrier()
        pltpu.sync_copy(acc_sh.at[pl.ds(sid * RPT, RPT)],
                        out_hbm.at[cid, pl.ds(sid * RPT, RPT)])

    return agg


_agg64 = _make_agg(D_HID)
_agg16 = _make_agg(D_OUT_PAD)


def _k2_body(deg_ref, x_ref, w1_ref, dinv_ref, h1_ref):
    deg = deg_ref[0, :N] + deg_ref[1, :N] + 1.0
    dinv = lax.rsqrt(deg)
    h = jnp.dot(x_ref[...], w1_ref[...],
                preferred_element_type=jnp.float32)
    dinv_ref[...] = dinv
    h1_ref[...] = h * dinv[:, None]


def _k4_body(s1_ref, h1_ref, dinv_ref, b1_ref, w2_ref, h2_ref):
    s = s1_ref[0, :N, :] + s1_ref[1, :N, :] + h1_ref[...]
    dinv = dinv_ref[...]
    o1 = jnp.maximum(s * dinv[:, None] + b1_ref[...][None, :], 0.0)
    h2_ref[...] = jnp.dot(o1 * dinv[:, None], w2_ref[...],
                          preferred_element_type=jnp.float32)


def _k6_body(s2_ref, h2_ref, dinv_ref, b2_ref, out_ref):
    s = s2_ref[0, :N, :D_OUT] + s2_ref[1, :N, :D_OUT] + h2_ref[:, :D_OUT]
    out_ref[...] = s * dinv_ref[...][:, None] + b2_ref[...][None, :]


def kernel(x, edge_index, W1, b1, W2, b2):
    ei3 = jnp.reshape(edge_index, (2, NCHUNKS, CH))
    z64 = jnp.zeros((RPT, D_HID), jnp.float32)
    z16 = jnp.zeros((RPT, D_OUT_PAD), jnp.float32)

    deg2 = _deg_kernel(ei3)

    dinv, h1 = pl.pallas_call(
        _k2_body,
        out_shape=[
            jax.ShapeDtypeStruct((N,), jnp.float32),
            jax.ShapeDtypeStruct((N, D_HID), jnp.float32),
        ],
    )(deg2, x, W1)

    s1 = _agg64(h1, ei3, z64)

    w2p = jnp.pad(W2, ((0, 0), (0, D_OUT_PAD - D_OUT)))
    h2 = pl.pallas_call(
        _k4_body,
        out_shape=jax.ShapeDtypeStruct((N, D_OUT_PAD), jnp.float32),
    )(s1, h1, dinv, b1, w2p)

    s2 = _agg16(h2, ei3, z16)

    out = pl.pallas_call(
        _k6_body,
        out_shape=jax.ShapeDtypeStruct((N, D_OUT), jnp.float32),
    )(s2, h2, dinv, b2)

    return out


# speedup vs baseline: 53.4693x; 1.1066x over previous
import functools

import jax
import jax.numpy as jnp
from jax import lax
from jax.experimental import pallas as pl
from jax.experimental.pallas import tpu as pltpu
from jax.experimental.pallas import tpu_sc as plsc

N = 10000
E = 320000
D_IN = 128
D_HID = 64
D_OUT = 10
D_OUT_PAD = 16

NC = 2
NS = 16
NW = NC * NS
CH = 128
NCHUNKS = E // CH
CPT = NCHUNKS // NW
REM = NCHUNKS - CPT * NW
NCH = CPT + 1

RPT = 632
N_ACC = RPT * NS
DPT = 640
N_DEG = DPT * NS


def _mesh():
    return plsc.VectorSubcoreMesh(core_axis_name="c", subcore_axis_name="s")


@functools.partial(
    pl.kernel,
    mesh=_mesh(),
    compiler_params=pltpu.CompilerParams(use_tc_tiling_on_sc=False),
    out_type=jax.ShapeDtypeStruct((NC * N_DEG,), jnp.float32),
    scratch_types=[
        pltpu.VMEM((NCH, CH), jnp.int32),
        pltpu.VMEM((CH,), jnp.float32),
        pltpu.VMEM((DPT,), jnp.float32),
        pltpu.VMEM_SHARED((N_DEG,), jnp.float32),
    ],
)
def _deg_kernel(ei3_hbm, out_hbm, idx_v, ones_v, zb_v, acc_sh):
    cid = lax.axis_index("c")
    sid = lax.axis_index("s")
    w = sid * NC + cid
    pltpu.sync_copy(ei3_hbm.at[1, pl.ds(w * CPT, CPT)],
                    idx_v.at[pl.ds(0, CPT)])

    @pl.when(w < REM)
    def _():
        pltpu.sync_copy(ei3_hbm.at[1, pl.ds(NW * CPT + w, 1)],
                        idx_v.at[pl.ds(CPT, 1)])

    nch = CPT + jnp.where(w < REM, 1, 0)
    for i in range(CH // 16):
        ones_v[pl.ds(i * 16, 16)] = jnp.ones((16,), jnp.float32)
    for i in range(DPT // 16):
        zb_v[pl.ds(i * 16, 16)] = jnp.zeros((16,), jnp.float32)
    pltpu.sync_copy(zb_v, acc_sh.at[pl.ds(sid * DPT, DPT)])
    plsc.subcore_barrier()

    def body(j, carry):
        pltpu.sync_copy(ones_v, acc_sh.at[idx_v.at[j]], add=True)
        return carry

    lax.fori_loop(0, nch, body, 0)
    plsc.subcore_barrier()
    pltpu.sync_copy(acc_sh.at[pl.ds(sid * DPT, DPT)],
                    out_hbm.at[pl.ds(cid * N_DEG + sid * DPT, DPT)])


def _make_agg(d):
    @functools.partial(
        pl.kernel,
        mesh=_mesh(),
        compiler_params=pltpu.CompilerParams(use_tc_tiling_on_sc=False),
        out_type=jax.ShapeDtypeStruct((NC, N_ACC, d), jnp.float32),
        scratch_types=[
            pltpu.VMEM((NCH, CH), jnp.int32),
            pltpu.VMEM((NCH, CH), jnp.int32),
            pltpu.VMEM((CH, d), jnp.float32),
            pltpu.VMEM((CH, d), jnp.float32),
            pltpu.VMEM((CH, d), jnp.float32),
            pltpu.VMEM((CH, d), jnp.float32),
            pltpu.VMEM_SHARED((N_ACC, d), jnp.float32),
            pltpu.SemaphoreType.DMA,
            pltpu.SemaphoreType.DMA,
            pltpu.SemaphoreType.DMA,
            pltpu.SemaphoreType.DMA,
            pltpu.SemaphoreType.DMA,
            pltpu.SemaphoreType.DMA,
            pltpu.SemaphoreType.DMA,
            pltpu.SemaphoreType.DMA,
        ],
    )
    def agg(h_hbm, ei3_hbm, zeros_hbm, out_hbm,
            src_v, dst_v, rows0, rows1, rows2, rows3, acc_sh,
            g0, g1, g2, g3, s0, s1, s2, s3):
        cid = lax.axis_index("c")
        sid = lax.axis_index("s")
        w = sid * NC + cid
        pltpu.sync_copy(ei3_hbm.at[0, pl.ds(w * CPT, CPT)],
                        src_v.at[pl.ds(0, CPT)])
        pltpu.sync_copy(ei3_hbm.at[1, pl.ds(w * CPT, CPT)],
                        dst_v.at[pl.ds(0, CPT)])

        @pl.when(w < REM)
        def _():
            pltpu.sync_copy(ei3_hbm.at[0, pl.ds(NW * CPT + w, 1)],
                            src_v.at[pl.ds(CPT, 1)])
            pltpu.sync_copy(ei3_hbm.at[1, pl.ds(NW * CPT + w, 1)],
                            dst_v.at[pl.ds(CPT, 1)])

        nch = CPT + jnp.where(w < REM, 1, 0)
        pltpu.sync_copy(zeros_hbm, acc_sh.at[pl.ds(sid * RPT, RPT)])
        plsc.subcore_barrier()

        rows = (rows0, rows1, rows2, rows3)
        gs = (g0, g1, g2, g3)
        ss = (s0, s1, s2, s3)
        for b in range(2):
            pltpu.async_copy(h_hbm.at[src_v.at[b]], rows[b], gs[b])

        def body(g, carry):
            for b in range(4):
                c = g * 4 + b

                @pl.when(c < nch)
                def _():
                    bn = (b + 2) % 4
                    pltpu.make_async_copy(
                        h_hbm.at[src_v.at[c]], rows[b], gs[b]).wait()
                    pltpu.sync_copy(rows[b], acc_sh.at[dst_v.at[c]], add=True)

                    @pl.when(c + 2 < nch)
                    def _():
                        pltpu.async_copy(
                            h_hbm.at[src_v.at[c + 2]], rows[bn], gs[bn])
            return carry

        lax.fori_loop(0, (NCH + 3) // 4, body, 0)

        plsc.subcore_barrier()
        pltpu.sync_copy(acc_sh.at[pl.ds(sid * RPT, RPT)],
                        out_hbm.at[cid, pl.ds(sid * RPT, RPT)])

    return agg


_agg64 = _make_agg(D_HID)
_agg16 = _make_agg(D_OUT_PAD)


def _k2a_body(deg_ref, dinv_ref):
    deg = deg_ref[pl.ds(0, N)] + deg_ref[pl.ds(N_DEG, N)] + 1.0
    dinv_ref[...] = lax.rsqrt(deg)


def _k2b_body(xp_ref, w1bd_ref, dp_ref, h1p_ref):
    h = jnp.dot(xp_ref[...], w1bd_ref[...],
                preferred_element_type=jnp.float32)
    h1p_ref[...] = h * dp_ref[...]


def _k4_body(s1p_ref, h1p_ref, dp_ref, b1p_ref, w2bd_ref, h2p_ref):
    s = s1p_ref[0, :N // 8, :] + s1p_ref[1, :N // 8, :] + h1p_ref[...]
    dp = dp_ref[...]
    o1 = jnp.maximum(s * dp + b1p_ref[...][None, :], 0.0)
    h2p_ref[...] = jnp.dot(o1 * dp, w2bd_ref[...],
                           preferred_element_type=jnp.float32)


def _k6_body(s2p_ref, h2p_ref, dp_ref, b2p_ref, outp_ref):
    s = s2p_ref[0, :N // 8, :] + s2p_ref[1, :N // 8, :] + h2p_ref[...]
    outp_ref[...] = s * dp_ref[...] + b2p_ref[...][None, :]


def kernel(x, edge_index, W1, b1, W2, b2):
    ei3 = jnp.reshape(edge_index, (2, NCHUNKS, CH))
    z64 = jnp.zeros((RPT, D_HID), jnp.float32)
    z16 = jnp.zeros((RPT, D_OUT_PAD), jnp.float32)

    deg2 = _deg_kernel(ei3)

    dinv = pl.pallas_call(
        _k2a_body,
        out_shape=jax.ShapeDtypeStruct((N,), jnp.float32),
    )(deg2)

    dp64 = jnp.reshape(jnp.broadcast_to(dinv[:, None], (N, D_HID)),
                       (N // 2, 128))
    dp16 = jnp.reshape(jnp.broadcast_to(dinv[:, None], (N, D_OUT_PAD)),
                       (N // 8, 128))
    w1bd = jnp.kron(jnp.eye(2, dtype=jnp.float32), W1)
    w2p = jnp.pad(W2, ((0, 0), (0, D_OUT_PAD - D_OUT)))
    w2bd = jnp.kron(jnp.eye(8, dtype=jnp.float32), w2p)
    b1p = jnp.tile(b1, 8)
    b2p = jnp.tile(jnp.pad(b2, (0, D_OUT_PAD - D_OUT)), 8)

    xp = jnp.reshape(x, (N // 2, 2 * D_IN))
    h1p2 = pl.pallas_call(
        _k2b_body,
        out_shape=jax.ShapeDtypeStruct((N // 2, 128), jnp.float32),
    )(xp, w1bd, dp64)

    s1 = _agg64(jnp.reshape(h1p2, (N, D_HID)), ei3, z64)

    h2p = pl.pallas_call(
        _k4_body,
        out_shape=jax.ShapeDtypeStruct((N // 8, 128), jnp.float32),
    )(jnp.reshape(s1, (NC, N_ACC // 8, 512)),
      jnp.reshape(h1p2, (N // 8, 512)),
      jnp.reshape(dp64, (N // 8, 512)),
      b1p, w2bd)

    s2 = _agg16(jnp.reshape(h2p, (N, D_OUT_PAD)), ei3, z16)

    outp = pl.pallas_call(
        _k6_body,
        out_shape=jax.ShapeDtypeStruct((N // 8, 128), jnp.float32),
    )(jnp.reshape(s2, (NC, N_ACC // 8, 128)), h2p, dp16, b2p)

    return jnp.reshape(outp, (N, D_OUT_PAD))[:, :D_OUT]


# speedup vs baseline: 56.1079x; 1.0493x over previous
import functools

import jax
import jax.numpy as jnp
from jax import lax
from jax.experimental import pallas as pl
from jax.experimental.pallas import tpu as pltpu
from jax.experimental.pallas import tpu_sc as plsc

N = 10000
E = 320000
D_IN = 128
D_HID = 64
D_OUT = 10
D_OUT_PAD = 16

NC = 2
NS = 16
NW = NC * NS
CH = 128
NCHUNKS = E // CH
CPT = NCHUNKS // NW
REM = NCHUNKS - CPT * NW
NCH = CPT + 1

RPT = 632
N_ACC = RPT * NS
DPT = 640
N_DEG = DPT * NS


def _mesh():
    return plsc.VectorSubcoreMesh(core_axis_name="c", subcore_axis_name="s")


@functools.partial(
    pl.kernel,
    mesh=_mesh(),
    compiler_params=pltpu.CompilerParams(use_tc_tiling_on_sc=False),
    out_type=jax.ShapeDtypeStruct((NC * N_DEG,), jnp.float32),
    scratch_types=[
        pltpu.VMEM((NCH, CH), jnp.int32),
        pltpu.VMEM((CH,), jnp.float32),
        pltpu.VMEM((DPT,), jnp.float32),
        pltpu.VMEM_SHARED((N_DEG,), jnp.float32),
    ],
)
def _deg_kernel(ei3_hbm, out_hbm, idx_v, ones_v, zb_v, acc_sh):
    cid = lax.axis_index("c")
    sid = lax.axis_index("s")
    w = sid * NC + cid
    pltpu.sync_copy(ei3_hbm.at[1, pl.ds(w * CPT, CPT)],
                    idx_v.at[pl.ds(0, CPT)])

    @pl.when(w < REM)
    def _():
        pltpu.sync_copy(ei3_hbm.at[1, pl.ds(NW * CPT + w, 1)],
                        idx_v.at[pl.ds(CPT, 1)])

    nch = CPT + jnp.where(w < REM, 1, 0)
    for i in range(CH // 16):
        ones_v[pl.ds(i * 16, 16)] = jnp.ones((16,), jnp.float32)
    for i in range(DPT // 16):
        zb_v[pl.ds(i * 16, 16)] = jnp.zeros((16,), jnp.float32)
    pltpu.sync_copy(zb_v, acc_sh.at[pl.ds(sid * DPT, DPT)])
    plsc.subcore_barrier()

    def body(j, carry):
        pltpu.sync_copy(ones_v, acc_sh.at[idx_v.at[j]], add=True)
        return carry

    lax.fori_loop(0, nch, body, 0)
    plsc.subcore_barrier()
    pltpu.sync_copy(acc_sh.at[pl.ds(sid * DPT, DPT)],
                    out_hbm.at[pl.ds(cid * N_DEG + sid * DPT, DPT)])


def _make_agg(d):
    @functools.partial(
        pl.kernel,
        mesh=_mesh(),
        compiler_params=pltpu.CompilerParams(use_tc_tiling_on_sc=False),
        out_type=jax.ShapeDtypeStruct((NC, N_ACC, d), jnp.float32),
        scratch_types=[
            pltpu.VMEM((NCH, CH), jnp.int32),
            pltpu.VMEM((NCH, CH), jnp.int32),
            pltpu.VMEM((CH, d), jnp.float32),
            pltpu.VMEM((CH, d), jnp.float32),
            pltpu.VMEM((CH, d), jnp.float32),
            pltpu.VMEM((CH, d), jnp.float32),
            pltpu.VMEM_SHARED((N_ACC, d), jnp.float32),
            pltpu.SemaphoreType.DMA,
            pltpu.SemaphoreType.DMA,
            pltpu.SemaphoreType.DMA,
            pltpu.SemaphoreType.DMA,
            pltpu.SemaphoreType.DMA,
            pltpu.SemaphoreType.DMA,
            pltpu.SemaphoreType.DMA,
            pltpu.SemaphoreType.DMA,
        ],
    )
    def agg(h_hbm, ei3_hbm, zeros_hbm, out_hbm,
            src_v, dst_v, rows0, rows1, rows2, rows3, acc_sh,
            g0, g1, g2, g3, s0, s1, s2, s3):
        cid = lax.axis_index("c")
        sid = lax.axis_index("s")
        w = sid * NC + cid
        pltpu.sync_copy(ei3_hbm.at[0, pl.ds(w * CPT, CPT)],
                        src_v.at[pl.ds(0, CPT)])
        pltpu.sync_copy(ei3_hbm.at[1, pl.ds(w * CPT, CPT)],
                        dst_v.at[pl.ds(0, CPT)])

        @pl.when(w < REM)
        def _():
            pltpu.sync_copy(ei3_hbm.at[0, pl.ds(NW * CPT + w, 1)],
                            src_v.at[pl.ds(CPT, 1)])
            pltpu.sync_copy(ei3_hbm.at[1, pl.ds(NW * CPT + w, 1)],
                            dst_v.at[pl.ds(CPT, 1)])

        nch = CPT + jnp.where(w < REM, 1, 0)
        pltpu.sync_copy(zeros_hbm, acc_sh.at[pl.ds(sid * RPT, RPT)])
        plsc.subcore_barrier()

        rows = (rows0, rows1, rows2, rows3)
        gs = (g0, g1, g2, g3)
        ss = (s0, s1, s2, s3)
        for b in range(2):
            pltpu.async_copy(h_hbm.at[src_v.at[b]], rows[b], gs[b])

        def body(g, carry):
            for b in range(4):
                c = g * 4 + b

                @pl.when(c < nch)
                def _():
                    bn = (b + 2) % 4
                    pltpu.make_async_copy(
                        h_hbm.at[src_v.at[c]], rows[b], gs[b]).wait()
                    pltpu.async_copy(
                        rows[b], acc_sh.at[dst_v.at[c]], ss[b], add=True)

                    @pl.when(c >= 2)
                    def _():
                        pltpu.make_async_copy(
                            rows[bn], acc_sh.at[dst_v.at[c - 2]],
                            ss[bn]).wait()

                    @pl.when(c + 2 < nch)
                    def _():
                        pltpu.async_copy(
                            h_hbm.at[src_v.at[c + 2]], rows[bn], gs[bn])
            return carry

        lax.fori_loop(0, (NCH + 3) // 4, body, 0)

        @pl.when(w < REM)
        def _():
            pltpu.make_async_copy(
                rows[1], acc_sh.at[dst_v.at[NCH - 2]], ss[1]).wait()
            pltpu.make_async_copy(
                rows[2], acc_sh.at[dst_v.at[NCH - 1]], ss[2]).wait()

        @pl.when(w >= REM)
        def _():
            pltpu.make_async_copy(
                rows[0], acc_sh.at[dst_v.at[CPT - 2]], ss[0]).wait()
            pltpu.make_async_copy(
                rows[1], acc_sh.at[dst_v.at[CPT - 1]], ss[1]).wait()

        plsc.subcore_barrier()
        pltpu.sync_copy(acc_sh.at[pl.ds(sid * RPT, RPT)],
                        out_hbm.at[cid, pl.ds(sid * RPT, RPT)])

    return agg


_agg64 = _make_agg(D_HID)
_agg16 = _make_agg(D_OUT_PAD)


def _k2a_body(deg_ref, dinv_ref):
    deg = deg_ref[pl.ds(0, N)] + deg_ref[pl.ds(N_DEG, N)] + 1.0
    dinv_ref[...] = lax.rsqrt(deg)


def _k2b_body(xp_ref, w1bd_ref, dp_ref, h1p_ref):
    h = jnp.dot(xp_ref[...], w1bd_ref[...],
                preferred_element_type=jnp.float32)
    h1p_ref[...] = h * dp_ref[...]


def _k4_body(s1p_ref, h1p_ref, dp_ref, b1p_ref, w2bd_ref, h2p_ref):
    s = s1p_ref[0, :N // 8, :] + s1p_ref[1, :N // 8, :] + h1p_ref[...]
    dp = dp_ref[...]
    o1 = jnp.maximum(s * dp + b1p_ref[...][None, :], 0.0)
    h2p_ref[...] = jnp.dot(o1 * dp, w2bd_ref[...],
                           preferred_element_type=jnp.float32)


def _k6_body(s2p_ref, h2p_ref, dp_ref, b2p_ref, outp_ref):
    s = s2p_ref[0, :N // 8, :] + s2p_ref[1, :N // 8, :] + h2p_ref[...]
    outp_ref[...] = s * dp_ref[...] + b2p_ref[...][None, :]


def kernel(x, edge_index, W1, b1, W2, b2):
    ei3 = jnp.reshape(edge_index, (2, NCHUNKS, CH))
    z64 = jnp.zeros((RPT, D_HID), jnp.float32)
    z16 = jnp.zeros((RPT, D_OUT_PAD), jnp.float32)

    deg2 = _deg_kernel(ei3)

    dinv = pl.pallas_call(
        _k2a_body,
        out_shape=jax.ShapeDtypeStruct((N,), jnp.float32),
    )(deg2)

    dp64 = jnp.reshape(jnp.broadcast_to(dinv[:, None], (N, D_HID)),
                       (N // 2, 128))
    dp16 = jnp.reshape(jnp.broadcast_to(dinv[:, None], (N, D_OUT_PAD)),
                       (N // 8, 128))
    w1bd = jnp.kron(jnp.eye(2, dtype=jnp.float32), W1)
    w2p = jnp.pad(W2, ((0, 0), (0, D_OUT_PAD - D_OUT)))
    w2bd = jnp.kron(jnp.eye(8, dtype=jnp.float32), w2p)
    b1p = jnp.tile(b1, 8)
    b2p = jnp.tile(jnp.pad(b2, (0, D_OUT_PAD - D_OUT)), 8)

    xp = jnp.reshape(x, (N // 2, 2 * D_IN))
    h1p2 = pl.pallas_call(
        _k2b_body,
        out_shape=jax.ShapeDtypeStruct((N // 2, 128), jnp.float32),
    )(xp, w1bd, dp64)

    s1 = _agg64(jnp.reshape(h1p2, (N, D_HID)), ei3, z64)

    h2p = pl.pallas_call(
        _k4_body,
        out_shape=jax.ShapeDtypeStruct((N // 8, 128), jnp.float32),
    )(jnp.reshape(s1, (NC, N_ACC // 8, 512)),
      jnp.reshape(h1p2, (N // 8, 512)),
      jnp.reshape(dp64, (N // 8, 512)),
      b1p, w2bd)

    s2 = _agg16(jnp.reshape(h2p, (N, D_OUT_PAD)), ei3, z16)

    outp = pl.pallas_call(
        _k6_body,
        out_shape=jax.ShapeDtypeStruct((N // 8, 128), jnp.float32),
    )(jnp.reshape(s2, (NC, N_ACC // 8, 128)), h2p, dp16, b2p)

    return jnp.reshape(outp, (N, D_OUT_PAD))[:, :D_OUT]


# speedup vs baseline: 56.7606x; 1.0116x over previous
import functools

import jax
import jax.numpy as jnp
from jax import lax
from jax.experimental import pallas as pl
from jax.experimental.pallas import tpu as pltpu
from jax.experimental.pallas import tpu_sc as plsc

N = 10000
E = 320000
D_IN = 128
D_HID = 64
D_OUT = 10
D_OUT_PAD = 16

NC = 2
NS = 16
NW = NC * NS
CH = 128
NCHUNKS = E // CH
CPT = NCHUNKS // NW
REM = NCHUNKS - CPT * NW
NCH = CPT + 1

RPT = 632
N_ACC = RPT * NS
DPT = 640
N_DEG = DPT * NS


def _mesh():
    return plsc.VectorSubcoreMesh(core_axis_name="c", subcore_axis_name="s")


@functools.partial(
    pl.kernel,
    mesh=_mesh(),
    compiler_params=pltpu.CompilerParams(use_tc_tiling_on_sc=False),
    out_type=jax.ShapeDtypeStruct((NC * N_DEG,), jnp.float32),
    scratch_types=[
        pltpu.VMEM((NCH, CH), jnp.int32),
        pltpu.VMEM((CH,), jnp.float32),
        pltpu.VMEM((DPT,), jnp.float32),
        pltpu.VMEM_SHARED((N_DEG,), jnp.float32),
    ],
)
def _deg_kernel(ei3_hbm, out_hbm, idx_v, ones_v, zb_v, acc_sh):
    cid = lax.axis_index("c")
    sid = lax.axis_index("s")
    w = sid * NC + cid
    pltpu.sync_copy(ei3_hbm.at[1, pl.ds(w * CPT, CPT)],
                    idx_v.at[pl.ds(0, CPT)])

    @pl.when(w < REM)
    def _():
        pltpu.sync_copy(ei3_hbm.at[1, pl.ds(NW * CPT + w, 1)],
                        idx_v.at[pl.ds(CPT, 1)])

    nch = CPT + jnp.where(w < REM, 1, 0)
    for i in range(CH // 16):
        ones_v[pl.ds(i * 16, 16)] = jnp.ones((16,), jnp.float32)
    for i in range(DPT // 16):
        zb_v[pl.ds(i * 16, 16)] = jnp.zeros((16,), jnp.float32)
    pltpu.sync_copy(zb_v, acc_sh.at[pl.ds(sid * DPT, DPT)])
    plsc.subcore_barrier()

    def body(j, carry):
        pltpu.sync_copy(ones_v, acc_sh.at[idx_v.at[j]], add=True)
        return carry

    lax.fori_loop(0, nch, body, 0)
    plsc.subcore_barrier()
    pltpu.sync_copy(acc_sh.at[pl.ds(sid * DPT, DPT)],
                    out_hbm.at[pl.ds(cid * N_DEG + sid * DPT, DPT)])


def _make_agg(d):
    @functools.partial(
        pl.kernel,
        mesh=_mesh(),
        compiler_params=pltpu.CompilerParams(use_tc_tiling_on_sc=False),
        out_type=jax.ShapeDtypeStruct((NC, N_ACC, d), jnp.float32),
        scratch_types=[
            pltpu.VMEM((NCH, CH), jnp.int32),
            pltpu.VMEM((NCH, CH), jnp.int32),
            pltpu.VMEM((CH, d), jnp.float32),
            pltpu.VMEM((CH, d), jnp.float32),
            pltpu.VMEM((CH, d), jnp.float32),
            pltpu.VMEM((CH, d), jnp.float32),
            pltpu.VMEM_SHARED((N_ACC, d), jnp.float32),
            pltpu.SemaphoreType.DMA,
            pltpu.SemaphoreType.DMA,
            pltpu.SemaphoreType.DMA,
            pltpu.SemaphoreType.DMA,
            pltpu.SemaphoreType.DMA,
            pltpu.SemaphoreType.DMA,
            pltpu.SemaphoreType.DMA,
            pltpu.SemaphoreType.DMA,
        ],
    )
    def agg(h_hbm, ei3_hbm, zeros_hbm, out_hbm,
            src_v, dst_v, rows0, rows1, rows2, rows3, acc_sh,
            g0, g1, g2, g3, s0, s1, s2, s3):
        cid = lax.axis_index("c")
        sid = lax.axis_index("s")
        w = sid * NC + cid
        pltpu.sync_copy(ei3_hbm.at[0, pl.ds(w * CPT, CPT)],
                        src_v.at[pl.ds(0, CPT)])
        pltpu.sync_copy(ei3_hbm.at[1, pl.ds(w * CPT, CPT)],
                        dst_v.at[pl.ds(0, CPT)])

        @pl.when(w < REM)
        def _():
            pltpu.sync_copy(ei3_hbm.at[0, pl.ds(NW * CPT + w, 1)],
                            src_v.at[pl.ds(CPT, 1)])
            pltpu.sync_copy(ei3_hbm.at[1, pl.ds(NW * CPT + w, 1)],
                            dst_v.at[pl.ds(CPT, 1)])

        nch = CPT + jnp.where(w < REM, 1, 0)
        pltpu.sync_copy(zeros_hbm, acc_sh.at[pl.ds(sid * RPT, RPT)])
        plsc.subcore_barrier()

        rows = (rows0, rows1, rows2, rows3)
        gs = (g0, g1, g2, g3)
        ss = (s0, s1, s2, s3)
        for b in range(2):
            pltpu.async_copy(h_hbm.at[src_v.at[b]], rows[b], gs[b])

        def body(g, carry):
            for b in range(4):
                c = g * 4 + b

                @pl.when(c < nch)
                def _():
                    bn = (b + 2) % 4
                    pltpu.make_async_copy(
                        h_hbm.at[src_v.at[c]], rows[b], gs[b]).wait()
                    pltpu.async_copy(
                        rows[b], acc_sh.at[dst_v.at[c]], ss[b], add=True)

                    @pl.when(c >= 2)
                    def _():
                        pltpu.make_async_copy(
                            rows[bn], acc_sh.at[dst_v.at[c - 2]],
                            ss[bn]).wait()

                    @pl.when(c + 2 < nch)
                    def _():
                        pltpu.async_copy(
                            h_hbm.at[src_v.at[c + 2]], rows[bn], gs[bn])
            return carry

        lax.fori_loop(0, (NCH + 3) // 4, body, 0)

        @pl.when(w < REM)
        def _():
            pltpu.make_async_copy(
                rows[1], acc_sh.at[dst_v.at[NCH - 2]], ss[1]).wait()
            pltpu.make_async_copy(
                rows[2], acc_sh.at[dst_v.at[NCH - 1]], ss[2]).wait()

        @pl.when(w >= REM)
        def _():
            pltpu.make_async_copy(
                rows[0], acc_sh.at[dst_v.at[CPT - 2]], ss[0]).wait()
            pltpu.make_async_copy(
                rows[1], acc_sh.at[dst_v.at[CPT - 1]], ss[1]).wait()

        plsc.subcore_barrier()
        pltpu.sync_copy(acc_sh.at[pl.ds(sid * RPT, RPT)],
                        out_hbm.at[cid, pl.ds(sid * RPT, RPT)])

    return agg


_agg64 = _make_agg(D_HID)
_agg16 = _make_agg(D_OUT_PAD)


def _k2a_body(deg_ref, dinv_ref):
    deg = deg_ref[pl.ds(0, N)] + deg_ref[pl.ds(N_DEG, N)] + 1.0
    dinv_ref[...] = lax.rsqrt(deg)


def _k2m_body(xp_ref, w1bd_ref, hup_ref):
    hup_ref[...] = jnp.dot(xp_ref[...], w1bd_ref[...],
                           preferred_element_type=jnp.float32)


def _k2s_body(hup_ref, dp_ref, h1p_ref):
    h1p_ref[...] = hup_ref[...] * dp_ref[...]


def _k4a_body(s1p_ref, h1p_ref, dp_ref, b1p_ref, o1d_ref):
    s = s1p_ref[0, :N // 2, :] + s1p_ref[1, :N // 2, :] + h1p_ref[...]
    dp = dp_ref[...]
    o1 = jnp.maximum(s * dp + b1p_ref[...][None, :], 0.0)
    o1d_ref[...] = o1 * dp


def _k4b_body(o1d8_ref, w2bd_ref, h2p_ref):
    h2p_ref[...] = jnp.dot(o1d8_ref[...], w2bd_ref[...],
                           preferred_element_type=jnp.float32)


def _k6_body(s2p_ref, h2p_ref, dp_ref, b2p_ref, outp_ref):
    s = s2p_ref[0, :N // 8, :] + s2p_ref[1, :N // 8, :] + h2p_ref[...]
    outp_ref[...] = s * dp_ref[...] + b2p_ref[...][None, :]


def kernel(x, edge_index, W1, b1, W2, b2):
    ei3 = jnp.reshape(edge_index, (2, NCHUNKS, CH))
    z64 = jnp.zeros((RPT, D_HID), jnp.float32)
    z16 = jnp.zeros((RPT, D_OUT_PAD), jnp.float32)

    deg2 = _deg_kernel(ei3)

    dinv = pl.pallas_call(
        _k2a_body,
        out_shape=jax.ShapeDtypeStruct((N,), jnp.float32),
    )(deg2)

    dp64 = jnp.reshape(jnp.broadcast_to(dinv[:, None], (N, D_HID)),
                       (N // 2, 128))
    dp16 = jnp.reshape(jnp.broadcast_to(dinv[:, None], (N, D_OUT_PAD)),
                       (N // 8, 128))
    w1bd = jnp.kron(jnp.eye(2, dtype=jnp.float32), W1)
    w2p = jnp.pad(W2, ((0, 0), (0, D_OUT_PAD - D_OUT)))
    w2bd = jnp.kron(jnp.eye(8, dtype=jnp.float32), w2p)
    b1p = jnp.tile(b1, 2)
    b2p = jnp.tile(jnp.pad(b2, (0, D_OUT_PAD - D_OUT)), 8)

    xp = jnp.reshape(x, (N // 2, 2 * D_IN))
    h1p2u = pl.pallas_call(
        _k2m_body,
        out_shape=jax.ShapeDtypeStruct((N // 2, 128), jnp.float32),
    )(xp, w1bd)
    h1p2 = pl.pallas_call(
        _k2s_body,
        out_shape=jax.ShapeDtypeStruct((N // 2, 128), jnp.float32),
    )(h1p2u, dp64)

    s1 = _agg64(jnp.reshape(h1p2, (N, D_HID)), ei3, z64)

    o1d2 = pl.pallas_call(
        _k4a_body,
        out_shape=jax.ShapeDtypeStruct((N // 2, 128), jnp.float32),
    )(jnp.reshape(s1, (NC, N_ACC // 2, 128)), h1p2, dp64, b1p)

    h2p = pl.pallas_call(
        _k4b_body,
        out_shape=jax.ShapeDtypeStruct((N // 8, 128), jnp.float32),
    )(jnp.reshape(o1d2, (N // 8, 512)), w2bd)

    s2 = _agg16(jnp.reshape(h2p, (N, D_OUT_PAD)), ei3, z16)

    outp = pl.pallas_call(
        _k6_body,
        out_shape=jax.ShapeDtypeStruct((N // 8, 128), jnp.float32),
    )(jnp.reshape(s2, (NC, N_ACC // 8, 128)), h2p, dp16, b2p)

    return jnp.reshape(outp, (N, D_OUT_PAD))[:, :D_OUT]


# speedup vs baseline: 64.7360x; 1.1405x over previous
import functools

import jax
import jax.numpy as jnp
from jax import lax
from jax.experimental import pallas as pl
from jax.experimental.pallas import tpu as pltpu
from jax.experimental.pallas import tpu_sc as plsc

N = 10000
E = 320000
D_IN = 128
D_HID = 64
D_OUT = 10
D_OUT_PAD = 16

NC = 2
NS = 16
NW = NC * NS
CH = 128
NCHUNKS = E // CH
CPT = NCHUNKS // NW
REM = NCHUNKS - CPT * NW
NCH = CPT + 1

RPT = 632
N_ACC = RPT * NS
DPT = 640
N_DEG = DPT * NS


def _mesh():
    return plsc.VectorSubcoreMesh(core_axis_name="c", subcore_axis_name="s")


@functools.partial(
    pl.kernel,
    mesh=_mesh(),
    compiler_params=pltpu.CompilerParams(use_tc_tiling_on_sc=False),
    out_type=jax.ShapeDtypeStruct((NC * N_DEG,), jnp.float32),
    scratch_types=[
        pltpu.VMEM((NCH, CH), jnp.int32),
        pltpu.VMEM((CH,), jnp.float32),
        pltpu.VMEM((DPT,), jnp.float32),
        pltpu.VMEM_SHARED((N_DEG,), jnp.float32),
    ],
)
def _deg_kernel(ei3_hbm, out_hbm, idx_v, ones_v, zb_v, acc_sh):
    cid = lax.axis_index("c")
    sid = lax.axis_index("s")
    w = sid * NC + cid
    pltpu.sync_copy(ei3_hbm.at[1, pl.ds(w * CPT, CPT)],
                    idx_v.at[pl.ds(0, CPT)])

    @pl.when(w < REM)
    def _():
        pltpu.sync_copy(ei3_hbm.at[1, pl.ds(NW * CPT + w, 1)],
                        idx_v.at[pl.ds(CPT, 1)])

    nch = CPT + jnp.where(w < REM, 1, 0)
    for i in range(CH // 16):
        ones_v[pl.ds(i * 16, 16)] = jnp.ones((16,), jnp.float32)
    for i in range(DPT // 16):
        zb_v[pl.ds(i * 16, 16)] = jnp.zeros((16,), jnp.float32)
    pltpu.sync_copy(zb_v, acc_sh.at[pl.ds(sid * DPT, DPT)])
    plsc.subcore_barrier()

    def body(j, carry):
        pltpu.sync_copy(ones_v, acc_sh.at[idx_v.at[j]], add=True)
        return carry

    lax.fori_loop(0, nch, body, 0)
    plsc.subcore_barrier()
    pltpu.sync_copy(acc_sh.at[pl.ds(sid * DPT, DPT)],
                    out_hbm.at[pl.ds(cid * N_DEG + sid * DPT, DPT)])


def _make_agg(d):
    @functools.partial(
        pl.kernel,
        mesh=_mesh(),
        compiler_params=pltpu.CompilerParams(use_tc_tiling_on_sc=False),
        out_type=jax.ShapeDtypeStruct((NC, N_ACC, d), jnp.float32),
        scratch_types=[
            pltpu.VMEM((NCH, CH), jnp.int32),
            pltpu.VMEM((NCH, CH), jnp.int32),
            pltpu.VMEM((CH, d), jnp.float32),
            pltpu.VMEM((CH, d), jnp.float32),
            pltpu.VMEM((CH, d), jnp.float32),
            pltpu.VMEM((CH, d), jnp.float32),
            pltpu.VMEM((CH, d), jnp.float32),
            pltpu.VMEM((CH, d), jnp.float32),
            pltpu.VMEM((CH, d), jnp.float32),
            pltpu.VMEM((CH, d), jnp.float32),
            pltpu.VMEM_SHARED((N_ACC, d), jnp.float32),
            pltpu.SemaphoreType.DMA,
            pltpu.SemaphoreType.DMA,
            pltpu.SemaphoreType.DMA,
            pltpu.SemaphoreType.DMA,
            pltpu.SemaphoreType.DMA,
            pltpu.SemaphoreType.DMA,
            pltpu.SemaphoreType.DMA,
            pltpu.SemaphoreType.DMA,
            pltpu.SemaphoreType.DMA,
            pltpu.SemaphoreType.DMA,
            pltpu.SemaphoreType.DMA,
            pltpu.SemaphoreType.DMA,
            pltpu.SemaphoreType.DMA,
            pltpu.SemaphoreType.DMA,
            pltpu.SemaphoreType.DMA,
            pltpu.SemaphoreType.DMA,
        ],
    )
    def agg(h_hbm, ei3_hbm, zeros_hbm, out_hbm,
            src_v, dst_v, rows0, rows1, rows2, rows3,
            rows4, rows5, rows6, rows7, acc_sh,
            g0, g1, g2, g3, g4, g5, g6, g7,
            s0, s1, s2, s3, s4, s5, s6, s7):
        cid = lax.axis_index("c")
        sid = lax.axis_index("s")
        w = sid * NC + cid
        pltpu.sync_copy(ei3_hbm.at[0, pl.ds(w * CPT, CPT)],
                        src_v.at[pl.ds(0, CPT)])
        pltpu.sync_copy(ei3_hbm.at[1, pl.ds(w * CPT, CPT)],
                        dst_v.at[pl.ds(0, CPT)])

        @pl.when(w < REM)
        def _():
            pltpu.sync_copy(ei3_hbm.at[0, pl.ds(NW * CPT + w, 1)],
                            src_v.at[pl.ds(CPT, 1)])
            pltpu.sync_copy(ei3_hbm.at[1, pl.ds(NW * CPT + w, 1)],
                            dst_v.at[pl.ds(CPT, 1)])

        nch = CPT + jnp.where(w < REM, 1, 0)
        pltpu.sync_copy(zeros_hbm, acc_sh.at[pl.ds(sid * RPT, RPT)])
        plsc.subcore_barrier()

        rows = (rows0, rows1, rows2, rows3, rows4, rows5, rows6, rows7)
        gs = (g0, g1, g2, g3, g4, g5, g6, g7)
        ss = (s0, s1, s2, s3, s4, s5, s6, s7)
        NB = 8
        LA = 4
        for b in range(LA):
            pltpu.async_copy(h_hbm.at[src_v.at[b]], rows[b], gs[b])

        def body(g, carry):
            for b in range(NB):
                c = g * NB + b

                @pl.when(c < nch)
                def _():
                    bn = (b + LA) % NB
                    pltpu.make_async_copy(
                        h_hbm.at[src_v.at[c]], rows[b], gs[b]).wait()
                    pltpu.async_copy(
                        rows[b], acc_sh.at[dst_v.at[c]], ss[b], add=True)

                    @pl.when(c >= LA)
                    def _():
                        pltpu.make_async_copy(
                            rows[bn], acc_sh.at[dst_v.at[c - LA]],
                            ss[bn]).wait()

                    @pl.when(c + LA < nch)
                    def _():
                        pltpu.async_copy(
                            h_hbm.at[src_v.at[c + LA]], rows[bn], gs[bn])
            return carry

        lax.fori_loop(0, (NCH + NB - 1) // NB, body, 0)

        @pl.when(w < REM)
        def _():
            for k in range(LA):
                cc = NCH - LA + k
                pltpu.make_async_copy(
                    rows[cc % NB], acc_sh.at[dst_v.at[cc]], ss[cc % NB]).wait()

        @pl.when(w >= REM)
        def _():
            for k in range(LA):
                cc = CPT - LA + k
                pltpu.make_async_copy(
                    rows[cc % NB], acc_sh.at[dst_v.at[cc]], ss[cc % NB]).wait()

        plsc.subcore_barrier()
        pltpu.sync_copy(acc_sh.at[pl.ds(sid * RPT, RPT)],
                        out_hbm.at[cid, pl.ds(sid * RPT, RPT)])

    return agg


_agg64 = _make_agg(D_HID)
_agg16 = _make_agg(D_OUT_PAD)


def _k2a_body(deg_ref, dinv_ref):
    deg = deg_ref[pl.ds(0, N)] + deg_ref[pl.ds(N_DEG, N)] + 1.0
    dinv_ref[...] = lax.rsqrt(deg)


def _k2m_body(xp_ref, w1bd_ref, hup_ref):
    hup_ref[...] = jnp.dot(xp_ref[...], w1bd_ref[...],
                           preferred_element_type=jnp.float32)


def _k2s_body(hup_ref, dp_ref, h1p_ref):
    h1p_ref[...] = hup_ref[...] * dp_ref[...]


def _k4a_body(s1p_ref, h1p_ref, dp_ref, b1p_ref, o1d_ref):
    s = s1p_ref[0, :N // 2, :] + s1p_ref[1, :N // 2, :] + h1p_ref[...]
    dp = dp_ref[...]
    o1 = jnp.maximum(s * dp + b1p_ref[...][None, :], 0.0)
    o1d_ref[...] = o1 * dp


def _k4b_body(o1d8_ref, w2bd_ref, h2p_ref):
    h2p_ref[...] = jnp.dot(o1d8_ref[...], w2bd_ref[...],
                           preferred_element_type=jnp.float32)


def _k6_body(s2p_ref, h2p_ref, dp_ref, b2p_ref, outp_ref):
    s = s2p_ref[0, :N // 8, :] + s2p_ref[1, :N // 8, :] + h2p_ref[...]
    outp_ref[...] = s * dp_ref[...] + b2p_ref[...][None, :]


def kernel(x, edge_index, W1, b1, W2, b2):
    ei3 = jnp.reshape(edge_index, (2, NCHUNKS, CH))
    z64 = jnp.zeros((RPT, D_HID), jnp.float32)
    z16 = jnp.zeros((RPT, D_OUT_PAD), jnp.float32)

    deg2 = _deg_kernel(ei3)

    dinv = pl.pallas_call(
        _k2a_body,
        out_shape=jax.ShapeDtypeStruct((N,), jnp.float32),
    )(deg2)

    dp64 = jnp.reshape(jnp.broadcast_to(dinv[:, None], (N, D_HID)),
                       (N // 2, 128))
    dp16 = jnp.reshape(jnp.broadcast_to(dinv[:, None], (N, D_OUT_PAD)),
                       (N // 8, 128))
    w1bd = jnp.kron(jnp.eye(2, dtype=jnp.float32), W1)
    w2p = jnp.pad(W2, ((0, 0), (0, D_OUT_PAD - D_OUT)))
    w2bd = jnp.kron(jnp.eye(8, dtype=jnp.float32), w2p)
    b1p = jnp.tile(b1, 2)
    b2p = jnp.tile(jnp.pad(b2, (0, D_OUT_PAD - D_OUT)), 8)

    xp = jnp.reshape(x, (N // 2, 2 * D_IN))
    h1p2u = pl.pallas_call(
        _k2m_body,
        out_shape=jax.ShapeDtypeStruct((N // 2, 128), jnp.float32),
    )(xp, w1bd)
    h1p2 = pl.pallas_call(
        _k2s_body,
        out_shape=jax.ShapeDtypeStruct((N // 2, 128), jnp.float32),
    )(h1p2u, dp64)

    s1 = _agg64(jnp.reshape(h1p2, (N, D_HID)), ei3, z64)

    o1d2 = pl.pallas_call(
        _k4a_body,
        out_shape=jax.ShapeDtypeStruct((N // 2, 128), jnp.float32),
    )(jnp.reshape(s1, (NC, N_ACC // 2, 128)), h1p2, dp64, b1p)

    h2p = pl.pallas_call(
        _k4b_body,
        out_shape=jax.ShapeDtypeStruct((N // 8, 128), jnp.float32),
    )(jnp.reshape(o1d2, (N // 8, 512)), w2bd)

    s2 = _agg16(jnp.reshape(h2p, (N, D_OUT_PAD)), ei3, z16)

    outp = pl.pallas_call(
        _k6_body,
        out_shape=jax.ShapeDtypeStruct((N // 8, 128), jnp.float32),
    )(jnp.reshape(s2, (NC, N_ACC // 8, 128)), h2p, dp16, b2p)

    return jnp.reshape(outp, (N, D_OUT_PAD))[:, :D_OUT]


# speedup vs baseline: 66.5445x; 1.0279x over previous
import functools

import jax
import jax.numpy as jnp
from jax import lax
from jax.experimental import pallas as pl
from jax.experimental.pallas import tpu as pltpu
from jax.experimental.pallas import tpu_sc as plsc

N = 10000
E = 320000
D_IN = 128
D_HID = 64
D_OUT = 10
D_OUT_PAD = 16

NC = 2
NS = 16
NW = NC * NS
CH = 128
NCHUNKS = E // CH
CPT = NCHUNKS // NW
REM = NCHUNKS - CPT * NW
NCH = CPT + 1

RPT = 632
N_ACC = RPT * NS
DPT = 640
N_DEG = DPT * NS


def _mesh():
    return plsc.VectorSubcoreMesh(core_axis_name="c", subcore_axis_name="s")


@functools.partial(
    pl.kernel,
    mesh=_mesh(),
    compiler_params=pltpu.CompilerParams(use_tc_tiling_on_sc=False),
    out_type=jax.ShapeDtypeStruct((NC * N_DEG,), jnp.float32),
    scratch_types=[
        pltpu.VMEM((NCH, CH), jnp.int32),
        pltpu.VMEM((CH,), jnp.float32),
        pltpu.VMEM((DPT,), jnp.float32),
        pltpu.VMEM_SHARED((N_DEG,), jnp.float32),
    ],
)
def _deg_kernel(ei3_hbm, out_hbm, idx_v, ones_v, zb_v, acc_sh):
    cid = lax.axis_index("c")
    sid = lax.axis_index("s")
    w = sid * NC + cid
    pltpu.sync_copy(ei3_hbm.at[1, pl.ds(w * CPT, CPT)],
                    idx_v.at[pl.ds(0, CPT)])

    @pl.when(w < REM)
    def _():
        pltpu.sync_copy(ei3_hbm.at[1, pl.ds(NW * CPT + w, 1)],
                        idx_v.at[pl.ds(CPT, 1)])

    nch = CPT + jnp.where(w < REM, 1, 0)
    for i in range(CH // 16):
        ones_v[pl.ds(i * 16, 16)] = jnp.ones((16,), jnp.float32)
    for i in range(DPT // 16):
        zb_v[pl.ds(i * 16, 16)] = jnp.zeros((16,), jnp.float32)
    pltpu.sync_copy(zb_v, acc_sh.at[pl.ds(sid * DPT, DPT)])
    plsc.subcore_barrier()

    def body(j, carry):
        pltpu.sync_copy(ones_v, acc_sh.at[idx_v.at[j]], add=True)
        return carry

    lax.fori_loop(0, nch, body, 0)
    plsc.subcore_barrier()
    pltpu.sync_copy(acc_sh.at[pl.ds(sid * DPT, DPT)],
                    out_hbm.at[pl.ds(cid * N_DEG + sid * DPT, DPT)])


def _make_agg(d, nb, la):
    scratch = (
        [pltpu.VMEM((NCH, CH), jnp.int32)] * 2
        + [pltpu.VMEM((CH, d), jnp.float32)] * nb
        + [pltpu.VMEM_SHARED((N_ACC, d), jnp.float32)]
        + [pltpu.SemaphoreType.DMA] * (2 * nb)
    )

    @functools.partial(
        pl.kernel,
        mesh=_mesh(),
        compiler_params=pltpu.CompilerParams(use_tc_tiling_on_sc=False),
        out_type=jax.ShapeDtypeStruct((NC, N_ACC, d), jnp.float32),
        scratch_types=scratch,
    )
    def agg(h_hbm, ei3_hbm, zeros_hbm, out_hbm, *scr):
        src_v, dst_v = scr[0], scr[1]
        rows = scr[2:2 + nb]
        acc_sh = scr[2 + nb]
        gs = scr[3 + nb:3 + 2 * nb]
        ss = scr[3 + 2 * nb:3 + 3 * nb]
        cid = lax.axis_index("c")
        sid = lax.axis_index("s")
        w = sid * NC + cid
        pltpu.sync_copy(ei3_hbm.at[0, pl.ds(w * CPT, CPT)],
                        src_v.at[pl.ds(0, CPT)])
        pltpu.sync_copy(ei3_hbm.at[1, pl.ds(w * CPT, CPT)],
                        dst_v.at[pl.ds(0, CPT)])

        @pl.when(w < REM)
        def _():
            pltpu.sync_copy(ei3_hbm.at[0, pl.ds(NW * CPT + w, 1)],
                            src_v.at[pl.ds(CPT, 1)])
            pltpu.sync_copy(ei3_hbm.at[1, pl.ds(NW * CPT + w, 1)],
                            dst_v.at[pl.ds(CPT, 1)])

        nch = CPT + jnp.where(w < REM, 1, 0)
        pltpu.sync_copy(zeros_hbm, acc_sh.at[pl.ds(sid * RPT, RPT)])
        plsc.subcore_barrier()

        for b in range(la):
            pltpu.async_copy(h_hbm.at[src_v.at[b]], rows[b], gs[b])

        def body(g, carry):
            for b in range(nb):
                c = g * nb + b

                @pl.when(c < nch)
                def _():
                    bn = (b + la) % nb
                    pltpu.make_async_copy(
                        h_hbm.at[src_v.at[c]], rows[b], gs[b]).wait()
                    pltpu.async_copy(
                        rows[b], acc_sh.at[dst_v.at[c]], ss[b], add=True)

                    @pl.when(c >= la)
                    def _():
                        pltpu.make_async_copy(
                            rows[bn], acc_sh.at[dst_v.at[c - la]],
                            ss[bn]).wait()

                    @pl.when(c + la < nch)
                    def _():
                        pltpu.async_copy(
                            h_hbm.at[src_v.at[c + la]], rows[bn], gs[bn])
            return carry

        lax.fori_loop(0, (NCH + nb - 1) // nb, body, 0)

        @pl.when(w < REM)
        def _():
            for k in range(la):
                cc = NCH - la + k
                pltpu.make_async_copy(
                    rows[cc % nb], acc_sh.at[dst_v.at[cc]], ss[cc % nb]).wait()

        @pl.when(w >= REM)
        def _():
            for k in range(la):
                cc = CPT - la + k
                pltpu.make_async_copy(
                    rows[cc % nb], acc_sh.at[dst_v.at[cc]], ss[cc % nb]).wait()

        plsc.subcore_barrier()
        pltpu.sync_copy(acc_sh.at[pl.ds(sid * RPT, RPT)],
                        out_hbm.at[cid, pl.ds(sid * RPT, RPT)])

    return agg


_agg64 = _make_agg(D_HID, 8, 4)
_agg16 = _make_agg(D_OUT_PAD, 12, 6)


def _k2a_body(deg_ref, dinv_ref):
    deg = deg_ref[pl.ds(0, N)] + deg_ref[pl.ds(N_DEG, N)] + 1.0
    dinv_ref[...] = lax.rsqrt(deg)


def _k2m_body(xp_ref, w1bd_ref, hup_ref):
    hup_ref[...] = jnp.dot(xp_ref[...], w1bd_ref[...],
                           preferred_element_type=jnp.float32)


def _k2s_body(hup_ref, dp_ref, h1p_ref):
    h1p_ref[...] = hup_ref[...] * dp_ref[...]


def _k4a_body(s1p_ref, h1p_ref, dp_ref, b1p_ref, o1d_ref):
    s = s1p_ref[0, :N // 2, :] + s1p_ref[1, :N // 2, :] + h1p_ref[...]
    dp = dp_ref[...]
    o1 = jnp.maximum(s * dp + b1p_ref[...][None, :], 0.0)
    o1d_ref[...] = o1 * dp


def _k4b_body(o1d8_ref, w2bd_ref, h2p_ref):
    h2p_ref[...] = jnp.dot(o1d8_ref[...], w2bd_ref[...],
                           preferred_element_type=jnp.float32)


def _k6_body(s2p_ref, h2p_ref, dp_ref, b2p_ref, outp_ref):
    s = s2p_ref[0, :N // 8, :] + s2p_ref[1, :N // 8, :] + h2p_ref[...]
    outp_ref[...] = s * dp_ref[...] + b2p_ref[...][None, :]


def kernel(x, edge_index, W1, b1, W2, b2):
    ei3 = jnp.reshape(edge_index, (2, NCHUNKS, CH))
    z64 = jnp.zeros((RPT, D_HID), jnp.float32)
    z16 = jnp.zeros((RPT, D_OUT_PAD), jnp.float32)

    deg2 = _deg_kernel(ei3)

    dinv = pl.pallas_call(
        _k2a_body,
        out_shape=jax.ShapeDtypeStruct((N,), jnp.float32),
    )(deg2)

    dp64 = jnp.reshape(jnp.broadcast_to(dinv[:, None], (N, D_HID)),
                       (N // 2, 128))
    dp16 = jnp.reshape(jnp.broadcast_to(dinv[:, None], (N, D_OUT_PAD)),
                       (N // 8, 128))
    w1bd = jnp.kron(jnp.eye(2, dtype=jnp.float32), W1)
    w2p = jnp.pad(W2, ((0, 0), (0, D_OUT_PAD - D_OUT)))
    w2bd = jnp.kron(jnp.eye(8, dtype=jnp.float32), w2p)
    b1p = jnp.tile(b1, 2)
    b2p = jnp.tile(jnp.pad(b2, (0, D_OUT_PAD - D_OUT)), 8)

    xp = jnp.reshape(x, (N // 2, 2 * D_IN))
    h1p2u = pl.pallas_call(
        _k2m_body,
        out_shape=jax.ShapeDtypeStruct((N // 2, 128), jnp.float32),
    )(xp, w1bd)
    h1p2 = pl.pallas_call(
        _k2s_body,
        out_shape=jax.ShapeDtypeStruct((N // 2, 128), jnp.float32),
    )(h1p2u, dp64)

    s1 = _agg64(jnp.reshape(h1p2, (N, D_HID)), ei3, z64)

    o1d2 = pl.pallas_call(
        _k4a_body,
        out_shape=jax.ShapeDtypeStruct((N // 2, 128), jnp.float32),
    )(jnp.reshape(s1, (NC, N_ACC // 2, 128)), h1p2, dp64, b1p)

    h2p = pl.pallas_call(
        _k4b_body,
        out_shape=jax.ShapeDtypeStruct((N // 8, 128), jnp.float32),
    )(jnp.reshape(o1d2, (N // 8, 512)), w2bd)

    s2 = _agg16(jnp.reshape(h2p, (N, D_OUT_PAD)), ei3, z16)

    outp = pl.pallas_call(
        _k6_body,
        out_shape=jax.ShapeDtypeStruct((N // 8, 128), jnp.float32),
    )(jnp.reshape(s2, (NC, N_ACC // 8, 128)), h2p, dp16, b2p)

    return jnp.reshape(outp, (N, D_OUT_PAD))[:, :D_OUT]


# speedup vs baseline: 67.3180x; 1.0116x over previous
import functools

import jax
import jax.numpy as jnp
from jax import lax
from jax.experimental import pallas as pl
from jax.experimental.pallas import tpu as pltpu
from jax.experimental.pallas import tpu_sc as plsc

N = 10000
E = 320000
D_IN = 128
D_HID = 64
D_OUT = 10
D_OUT_PAD = 16

NC = 2
NS = 16
NW = NC * NS
CH = 128
NCHUNKS = E // CH
CPT = NCHUNKS // NW
REM = NCHUNKS - CPT * NW
NCH = CPT + 1

RPT = 632
N_ACC = RPT * NS
DPT = 640
N_DEG = DPT * NS


def _mesh():
    return plsc.VectorSubcoreMesh(core_axis_name="c", subcore_axis_name="s")


@functools.partial(
    pl.kernel,
    mesh=_mesh(),
    compiler_params=pltpu.CompilerParams(use_tc_tiling_on_sc=False),
    out_type=jax.ShapeDtypeStruct((NC * N_DEG,), jnp.float32),
    scratch_types=[
        pltpu.VMEM((NCH, CH), jnp.int32),
        pltpu.VMEM((CH,), jnp.float32),
        pltpu.VMEM((DPT,), jnp.float32),
        pltpu.VMEM_SHARED((N_DEG,), jnp.float32),
        pltpu.SemaphoreType.DMA,
        pltpu.SemaphoreType.DMA,
        pltpu.SemaphoreType.DMA,
        pltpu.SemaphoreType.DMA,
    ],
)
def _deg_kernel(ei3_hbm, out_hbm, idx_v, ones_v, zb_v, acc_sh,
                d0, d1, d2, d3):
    cid = lax.axis_index("c")
    sid = lax.axis_index("s")
    w = sid * NC + cid
    pltpu.sync_copy(ei3_hbm.at[1, pl.ds(w * CPT, CPT)],
                    idx_v.at[pl.ds(0, CPT)])

    @pl.when(w < REM)
    def _():
        pltpu.sync_copy(ei3_hbm.at[1, pl.ds(NW * CPT + w, 1)],
                        idx_v.at[pl.ds(CPT, 1)])

    nch = CPT + jnp.where(w < REM, 1, 0)
    for i in range(CH // 16):
        ones_v[pl.ds(i * 16, 16)] = jnp.ones((16,), jnp.float32)
    for i in range(DPT // 16):
        zb_v[pl.ds(i * 16, 16)] = jnp.zeros((16,), jnp.float32)
    pltpu.sync_copy(zb_v, acc_sh.at[pl.ds(sid * DPT, DPT)])
    plsc.subcore_barrier()

    dsem = (d0, d1, d2, d3)

    def body(g, carry):
        for b in range(4):
            c = g * 4 + b

            @pl.when(c < nch)
            def _():
                @pl.when(c >= 4)
                def _():
                    pltpu.make_async_copy(
                        ones_v, acc_sh.at[idx_v.at[c - 4]], dsem[b]).wait()

                pltpu.async_copy(
                    ones_v, acc_sh.at[idx_v.at[c]], dsem[b], add=True)
        return carry

    lax.fori_loop(0, (NCH + 3) // 4, body, 0)

    @pl.when(w < REM)
    def _():
        for k in range(4):
            cc = NCH - 4 + k
            pltpu.make_async_copy(
                ones_v, acc_sh.at[idx_v.at[cc]], dsem[cc % 4]).wait()

    @pl.when(w >= REM)
    def _():
        for k in range(4):
            cc = CPT - 4 + k
            pltpu.make_async_copy(
                ones_v, acc_sh.at[idx_v.at[cc]], dsem[cc % 4]).wait()

    plsc.subcore_barrier()
    pltpu.sync_copy(acc_sh.at[pl.ds(sid * DPT, DPT)],
                    out_hbm.at[pl.ds(cid * N_DEG + sid * DPT, DPT)])


def _make_agg(d, nb, la):
    scratch = (
        [pltpu.VMEM((NCH, CH), jnp.int32)] * 2
        + [pltpu.VMEM((CH, d), jnp.float32)] * nb
        + [pltpu.VMEM_SHARED((N_ACC, d), jnp.float32)]
        + [pltpu.SemaphoreType.DMA] * (2 * nb)
    )

    @functools.partial(
        pl.kernel,
        mesh=_mesh(),
        compiler_params=pltpu.CompilerParams(use_tc_tiling_on_sc=False),
        out_type=jax.ShapeDtypeStruct((NC, N_ACC, d), jnp.float32),
        scratch_types=scratch,
    )
    def agg(h_hbm, ei3_hbm, zeros_hbm, out_hbm, *scr):
        src_v, dst_v = scr[0], scr[1]
        rows = scr[2:2 + nb]
        acc_sh = scr[2 + nb]
        gs = scr[3 + nb:3 + 2 * nb]
        ss = scr[3 + 2 * nb:3 + 3 * nb]
        cid = lax.axis_index("c")
        sid = lax.axis_index("s")
        w = sid * NC + cid
        pltpu.sync_copy(ei3_hbm.at[0, pl.ds(w * CPT, CPT)],
                        src_v.at[pl.ds(0, CPT)])
        pltpu.sync_copy(ei3_hbm.at[1, pl.ds(w * CPT, CPT)],
                        dst_v.at[pl.ds(0, CPT)])

        @pl.when(w < REM)
        def _():
            pltpu.sync_copy(ei3_hbm.at[0, pl.ds(NW * CPT + w, 1)],
                            src_v.at[pl.ds(CPT, 1)])
            pltpu.sync_copy(ei3_hbm.at[1, pl.ds(NW * CPT + w, 1)],
                            dst_v.at[pl.ds(CPT, 1)])

        nch = CPT + jnp.where(w < REM, 1, 0)
        pltpu.sync_copy(zeros_hbm, acc_sh.at[pl.ds(sid * RPT, RPT)])
        plsc.subcore_barrier()

        for b in range(la):
            pltpu.async_copy(h_hbm.at[src_v.at[b]], rows[b], gs[b])

        def body(g, carry):
            for b in range(nb):
                c = g * nb + b

                @pl.when(c < nch)
                def _():
                    bn = (b + la) % nb
                    pltpu.make_async_copy(
                        h_hbm.at[src_v.at[c]], rows[b], gs[b]).wait()
                    pltpu.async_copy(
                        rows[b], acc_sh.at[dst_v.at[c]], ss[b], add=True)

                    @pl.when(c >= la)
                    def _():
                        pltpu.make_async_copy(
                            rows[bn], acc_sh.at[dst_v.at[c - la]],
                            ss[bn]).wait()

                    @pl.when(c + la < nch)
                    def _():
                        pltpu.async_copy(
                            h_hbm.at[src_v.at[c + la]], rows[bn], gs[bn])
            return carry

        lax.fori_loop(0, (NCH + nb - 1) // nb, body, 0)

        @pl.when(w < REM)
        def _():
            for k in range(la):
                cc = NCH - la + k
                pltpu.make_async_copy(
                    rows[cc % nb], acc_sh.at[dst_v.at[cc]], ss[cc % nb]).wait()

        @pl.when(w >= REM)
        def _():
            for k in range(la):
                cc = CPT - la + k
                pltpu.make_async_copy(
                    rows[cc % nb], acc_sh.at[dst_v.at[cc]], ss[cc % nb]).wait()

        plsc.subcore_barrier()
        pltpu.sync_copy(acc_sh.at[pl.ds(sid * RPT, RPT)],
                        out_hbm.at[cid, pl.ds(sid * RPT, RPT)])

    return agg


_agg64 = _make_agg(D_HID, 8, 4)
_agg16 = _make_agg(D_OUT_PAD, 12, 6)


def _k2a_body(deg_ref, dinv_ref):
    deg = deg_ref[pl.ds(0, N)] + deg_ref[pl.ds(N_DEG, N)] + 1.0
    dinv_ref[...] = lax.rsqrt(deg)


def _k2m_body(xp_ref, w1bd_ref, hup_ref):
    hup_ref[...] = jnp.dot(xp_ref[...], w1bd_ref[...],
                           preferred_element_type=jnp.float32)


def _k2s_body(hup_ref, dp_ref, h1p_ref):
    h1p_ref[...] = hup_ref[...] * dp_ref[...]


def _k4a_body(s1p_ref, h1p_ref, dp_ref, b1p_ref, o1d_ref):
    s = s1p_ref[0, :N // 2, :] + s1p_ref[1, :N // 2, :] + h1p_ref[...]
    dp = dp_ref[...]
    o1 = jnp.maximum(s * dp + b1p_ref[...][None, :], 0.0)
    o1d_ref[...] = o1 * dp


def _k4b_body(o1d8_ref, w2bd_ref, h2p_ref):
    h2p_ref[...] = jnp.dot(o1d8_ref[...], w2bd_ref[...],
                           preferred_element_type=jnp.float32)


def _k6_body(s2p_ref, h2p_ref, dp_ref, b2p_ref, outp_ref):
    s = s2p_ref[0, :N // 8, :] + s2p_ref[1, :N // 8, :] + h2p_ref[...]
    outp_ref[...] = s * dp_ref[...] + b2p_ref[...][None, :]


def kernel(x, edge_index, W1, b1, W2, b2):
    ei3 = jnp.reshape(edge_index, (2, NCHUNKS, CH))
    z64 = jnp.zeros((RPT, D_HID), jnp.float32)
    z16 = jnp.zeros((RPT, D_OUT_PAD), jnp.float32)

    deg2 = _deg_kernel(ei3)

    dinv = pl.pallas_call(
        _k2a_body,
        out_shape=jax.ShapeDtypeStruct((N,), jnp.float32),
    )(deg2)

    dp64 = jnp.reshape(jnp.broadcast_to(dinv[:, None], (N, D_HID)),
                       (N // 2, 128))
    dp16 = jnp.reshape(jnp.broadcast_to(dinv[:, None], (N, D_OUT_PAD)),
                       (N // 8, 128))
    w1bd = jnp.kron(jnp.eye(2, dtype=jnp.float32), W1)
    w2p = jnp.pad(W2, ((0, 0), (0, D_OUT_PAD - D_OUT)))
    w2bd = jnp.kron(jnp.eye(8, dtype=jnp.float32), w2p)
    b1p = jnp.tile(b1, 2)
    b2p = jnp.tile(jnp.pad(b2, (0, D_OUT_PAD - D_OUT)), 8)

    xp = jnp.reshape(x, (N // 2, 2 * D_IN))
    h1p2u = pl.pallas_call(
        _k2m_body,
        out_shape=jax.ShapeDtypeStruct((N // 2, 128), jnp.float32),
    )(xp, w1bd)
    h1p2 = pl.pallas_call(
        _k2s_body,
        out_shape=jax.ShapeDtypeStruct((N // 2, 128), jnp.float32),
    )(h1p2u, dp64)

    s1 = _agg64(jnp.reshape(h1p2, (N, D_HID)), ei3, z64)

    o1d2 = pl.pallas_call(
        _k4a_body,
        out_shape=jax.ShapeDtypeStruct((N // 2, 128), jnp.float32),
    )(jnp.reshape(s1, (NC, N_ACC // 2, 128)), h1p2, dp64, b1p)

    h2p = pl.pallas_call(
        _k4b_body,
        out_shape=jax.ShapeDtypeStruct((N // 8, 128), jnp.float32),
    )(jnp.reshape(o1d2, (N // 8, 512)), w2bd)

    s2 = _agg16(jnp.reshape(h2p, (N, D_OUT_PAD)), ei3, z16)

    outp = pl.pallas_call(
        _k6_body,
        out_shape=jax.ShapeDtypeStruct((N // 8, 128), jnp.float32),
    )(jnp.reshape(s2, (NC, N_ACC // 8, 128)), h2p, dp16, b2p)

    return jnp.reshape(outp, (N, D_OUT_PAD))[:, :D_OUT]
